# baseline (speedup 1.0000x reference)
import functools

import jax
import jax.numpy as jnp
from jax.experimental import pallas as pl
from jax.experimental.pallas import tpu as pltpu


def _proj_body(meth_ref, w1_ref, fc1w_ref, fc1b_ref, h0_ref, init_ref):
    x = meth_ref[0]
    h0_ref[0] = jnp.dot(x, w1_ref[...], preferred_element_type=jnp.float32)
    init_ref[0] = (
        jnp.dot(x, fc1w_ref[...], preferred_element_type=jnp.float32)
        + fc1b_ref[...]
    )


def _pass1_body(adj_ref, h0_ref, w2_ref, b1_ref, g_ref):
    a = adj_ref[0]
    h1 = jnp.maximum(
        jnp.dot(a, h0_ref[0], preferred_element_type=jnp.float32)
        + b1_ref[...],
        0.0,
    )
    g_ref[0] = jnp.dot(h1, w2_ref[...], preferred_element_type=jnp.float32)


def _pass2_body(adj_ref, g_ref, init_ref, b2_ref, pool_ref, *, tm, n):
    t = pl.program_id(1)
    a = adj_ref[0]
    o = (
        jnp.dot(a, g_ref[0], preferred_element_type=jnp.float32)
        + init_ref[0]
        + b2_ref[...]
    )
    rows = t * tm + jax.lax.broadcasted_iota(jnp.int32, o.shape, 0)
    o = jnp.where(rows < n, o, -jnp.inf)
    tile_max = jnp.max(o, axis=0, keepdims=True)

    @pl.when(t == 0)
    def _init():
        pool_ref[0] = tile_max

    @pl.when(t != 0)
    def _acc():
        pool_ref[0] = jnp.maximum(pool_ref[0], tile_max)


def _branch(adj, meth, w1, b1, w2, b2, fc1w, fc1b):
    bsz, n, f = meth.shape
    h = w1.shape[1]
    c = w2.shape[1]
    n_tiles = 4
    tm = (-(-n // n_tiles) + 7) // 8 * 8

    h0, init = pl.pallas_call(
        _proj_body,
        grid=(bsz,),
        in_specs=[
            pl.BlockSpec((1, n, f), lambda b: (b, 0, 0)),
            pl.BlockSpec((f, h), lambda b: (0, 0)),
            pl.BlockSpec((f, c), lambda b: (0, 0)),
            pl.BlockSpec((1, c), lambda b: (0, 0)),
        ],
        out_specs=[
            pl.BlockSpec((1, n, h), lambda b: (b, 0, 0)),
            pl.BlockSpec((1, n, c), lambda b: (b, 0, 0)),
        ],
        out_shape=[
            jax.ShapeDtypeStruct((bsz, n, h), jnp.float32),
            jax.ShapeDtypeStruct((bsz, n, c), jnp.float32),
        ],
        compiler_params=pltpu.CompilerParams(
            dimension_semantics=("parallel",),
        ),
    )(meth, w1, fc1w, fc1b)

    g = pl.pallas_call(
        _pass1_body,
        grid=(bsz, n_tiles),
        in_specs=[
            pl.BlockSpec((1, tm, n), lambda b, t: (b, t, 0)),
            pl.BlockSpec((1, n, h), lambda b, t: (b, 0, 0)),
            pl.BlockSpec((h, c), lambda b, t: (0, 0)),
            pl.BlockSpec((1, h), lambda b, t: (0, 0)),
        ],
        out_specs=pl.BlockSpec((1, tm, c), lambda b, t: (b, t, 0)),
        out_shape=jax.ShapeDtypeStruct((bsz, n, c), jnp.float32),
        compiler_params=pltpu.CompilerParams(
            dimension_semantics=("parallel", "arbitrary"),
        ),
    )(adj, h0, w2, b1)

    pool = pl.pallas_call(
        functools.partial(_pass2_body, tm=tm, n=n),
        grid=(bsz, n_tiles),
        in_specs=[
            pl.BlockSpec((1, tm, n), lambda b, t: (b, t, 0)),
            pl.BlockSpec((1, n, c), lambda b, t: (b, 0, 0)),
            pl.BlockSpec((1, tm, c), lambda b, t: (b, t, 0)),
            pl.BlockSpec((1, c), lambda b, t: (0, 0)),
        ],
        out_specs=pl.BlockSpec((1, 1, c), lambda b, t: (b, 0, 0)),
        out_shape=jax.ShapeDtypeStruct((bsz, 1, c), jnp.float32),
        compiler_params=pltpu.CompilerParams(
            dimension_semantics=("parallel", "arbitrary"),
        ),
    )(adj, g, init, b2)

    return pool.reshape(bsz, c)


def _mlp_body(sp_ref, vp_ref, w2a_ref, w2b_ref, b2_ref, w3_ref, b3_ref,
              w4_ref, b4_ref, w5_ref, b5_ref, out_ref):
    d = jnp.maximum(
        jnp.dot(sp_ref[...], w2a_ref[...], preferred_element_type=jnp.float32)
        + jnp.dot(vp_ref[...], w2b_ref[...], preferred_element_type=jnp.float32)
        + b2_ref[...],
        0.0,
    )
    d = jnp.maximum(
        jnp.dot(d, w3_ref[...], preferred_element_type=jnp.float32)
        + b3_ref[...],
        0.0,
    )
    d = jnp.maximum(
        jnp.dot(d, w4_ref[...], preferred_element_type=jnp.float32)
        + b4_ref[...],
        0.0,
    )
    out_ref[...] = (
        jnp.sum(d * w5_ref[...].T, axis=1, keepdims=True) + b5_ref[...]
    )


def kernel(solute_adj, solute_meth, solvent_meth, solvent_adj_meth,
           conv1_W, conv1_b, conv2_W, conv2_b,
           fc1_W, fc1_b, fc2_W, fc2_b, fc3_W, fc3_b,
           fc4_W, fc4_b, fc5_W, fc5_b):
    b1 = conv1_b.reshape(1, -1)
    b2 = conv2_b.reshape(1, -1)
    fb1 = fc1_b.reshape(1, -1)
    nclass = fc1_W.shape[1]

    su_pool = _branch(solute_adj, solute_meth, conv1_W, b1, conv2_W, b2,
                      fc1_W, fb1)
    sv_pool = _branch(solvent_adj_meth, solvent_meth, conv1_W, b1, conv2_W,
                      b2, fc1_W, fb1)

    bsz = su_pool.shape[0]
    out = pl.pallas_call(
        _mlp_body,
        out_shape=jax.ShapeDtypeStruct((bsz, 1), jnp.float32),
    )(su_pool, sv_pool,
      fc2_W[:nclass], fc2_W[nclass:], fc2_b.reshape(1, -1),
      fc3_W, fc3_b.reshape(1, -1),
      fc4_W, fc4_b.reshape(1, -1),
      fc5_W, fc5_b.reshape(1, -1))
    return out


# speedup vs baseline: 1.0149x; 1.0149x over previous
import functools

import jax
import jax.numpy as jnp
from jax.experimental import pallas as pl
from jax.experimental.pallas import tpu as pltpu


def _proj_body(meth_ref, w1_ref, fc1w_ref, fc1b_ref, h0_ref, init_ref):
    x = meth_ref[0]
    h0_ref[0] = (
        jnp.dot(x, w1_ref[...], preferred_element_type=jnp.float32)
        .astype(jnp.bfloat16)
    )
    init_ref[0] = (
        jnp.dot(x, fc1w_ref[...], preferred_element_type=jnp.float32)
        + fc1b_ref[...]
    )


def _pass1_body(adj_ref, h0_ref, w2_ref, b1_ref, g_ref):
    a = adj_ref[0].astype(jnp.bfloat16)
    h1 = jnp.maximum(
        jnp.dot(a, h0_ref[0], preferred_element_type=jnp.float32)
        + b1_ref[...],
        0.0,
    ).astype(jnp.bfloat16)
    g_ref[0] = (
        jnp.dot(h1, w2_ref[...].astype(jnp.bfloat16),
                preferred_element_type=jnp.float32)
        .astype(jnp.bfloat16)
    )


def _pass2_body(adj_ref, g_ref, init_ref, b2_ref, pool_ref, *, tm, n):
    t = pl.program_id(1)
    a = adj_ref[0].astype(jnp.bfloat16)
    o = (
        jnp.dot(a, g_ref[0], preferred_element_type=jnp.float32)
        + init_ref[0]
        + b2_ref[...]
    )
    rows = t * tm + jax.lax.broadcasted_iota(jnp.int32, o.shape, 0)
    o = jnp.where(rows < n, o, -jnp.inf)
    tile_max = jnp.max(o, axis=0, keepdims=True)

    @pl.when(t == 0)
    def _init():
        pool_ref[0] = tile_max

    @pl.when(t != 0)
    def _acc():
        pool_ref[0] = jnp.maximum(pool_ref[0], tile_max)


def _branch(adj, meth, w1, b1, w2, b2, fc1w, fc1b):
    bsz, n, f = meth.shape
    h = w1.shape[1]
    c = w2.shape[1]
    n_tiles = 4
    tm = (-(-n // n_tiles) + 7) // 8 * 8

    h0, init = pl.pallas_call(
        _proj_body,
        grid=(bsz,),
        in_specs=[
            pl.BlockSpec((1, n, f), lambda b: (b, 0, 0)),
            pl.BlockSpec((f, h), lambda b: (0, 0)),
            pl.BlockSpec((f, c), lambda b: (0, 0)),
            pl.BlockSpec((1, c), lambda b: (0, 0)),
        ],
        out_specs=[
            pl.BlockSpec((1, n, h), lambda b: (b, 0, 0)),
            pl.BlockSpec((1, n, c), lambda b: (b, 0, 0)),
        ],
        out_shape=[
            jax.ShapeDtypeStruct((bsz, n, h), jnp.bfloat16),
            jax.ShapeDtypeStruct((bsz, n, c), jnp.float32),
        ],
        compiler_params=pltpu.CompilerParams(
            dimension_semantics=("parallel",),
        ),
    )(meth, w1, fc1w, fc1b)

    g = pl.pallas_call(
        _pass1_body,
        grid=(bsz, n_tiles),
        in_specs=[
            pl.BlockSpec((1, tm, n), lambda b, t: (b, t, 0)),
            pl.BlockSpec((1, n, h), lambda b, t: (b, 0, 0)),
            pl.BlockSpec((h, c), lambda b, t: (0, 0)),
            pl.BlockSpec((1, h), lambda b, t: (0, 0)),
        ],
        out_specs=pl.BlockSpec((1, tm, c), lambda b, t: (b, t, 0)),
        out_shape=jax.ShapeDtypeStruct((bsz, n, c), jnp.bfloat16),
        compiler_params=pltpu.CompilerParams(
            dimension_semantics=("parallel", "arbitrary"),
        ),
    )(adj, h0, w2, b1)

    pool = pl.pallas_call(
        functools.partial(_pass2_body, tm=tm, n=n),
        grid=(bsz, n_tiles),
        in_specs=[
            pl.BlockSpec((1, tm, n), lambda b, t: (b, t, 0)),
            pl.BlockSpec((1, n, c), lambda b, t: (b, 0, 0)),
            pl.BlockSpec((1, tm, c), lambda b, t: (b, t, 0)),
            pl.BlockSpec((1, c), lambda b, t: (0, 0)),
        ],
        out_specs=pl.BlockSpec((1, 1, c), lambda b, t: (b, 0, 0)),
        out_shape=jax.ShapeDtypeStruct((bsz, 1, c), jnp.float32),
        compiler_params=pltpu.CompilerParams(
            dimension_semantics=("parallel", "arbitrary"),
        ),
    )(adj, g, init, b2)

    return pool.reshape(bsz, c)


def _mlp_body(sp_ref, vp_ref, w2a_ref, w2b_ref, b2_ref, w3_ref, b3_ref,
              w4_ref, b4_ref, w5_ref, b5_ref, out_ref):
    d = jnp.maximum(
        jnp.dot(sp_ref[...], w2a_ref[...], preferred_element_type=jnp.float32)
        + jnp.dot(vp_ref[...], w2b_ref[...], preferred_element_type=jnp.float32)
        + b2_ref[...],
        0.0,
    )
    d = jnp.maximum(
        jnp.dot(d, w3_ref[...], preferred_element_type=jnp.float32)
        + b3_ref[...],
        0.0,
    )
    d = jnp.maximum(
        jnp.dot(d, w4_ref[...], preferred_element_type=jnp.float32)
        + b4_ref[...],
        0.0,
    )
    out_ref[...] = (
        jnp.sum(d * w5_ref[...].T, axis=1, keepdims=True) + b5_ref[...]
    )


def kernel(solute_adj, solute_meth, solvent_meth, solvent_adj_meth,
           conv1_W, conv1_b, conv2_W, conv2_b,
           fc1_W, fc1_b, fc2_W, fc2_b, fc3_W, fc3_b,
           fc4_W, fc4_b, fc5_W, fc5_b):
    b1 = conv1_b.reshape(1, -1)
    b2 = conv2_b.reshape(1, -1)
    fb1 = fc1_b.reshape(1, -1)
    nclass = fc1_W.shape[1]

    su_pool = _branch(solute_adj, solute_meth, conv1_W, b1, conv2_W, b2,
                      fc1_W, fb1)
    sv_pool = _branch(solvent_adj_meth, solvent_meth, conv1_W, b1, conv2_W,
                      b2, fc1_W, fb1)

    bsz = su_pool.shape[0]
    out = pl.pallas_call(
        _mlp_body,
        out_shape=jax.ShapeDtypeStruct((bsz, 1), jnp.float32),
    )(su_pool, sv_pool,
      fc2_W[:nclass], fc2_W[nclass:], fc2_b.reshape(1, -1),
      fc3_W, fc3_b.reshape(1, -1),
      fc4_W, fc4_b.reshape(1, -1),
      fc5_W, fc5_b.reshape(1, -1))
    return out


# speedup vs baseline: 1.2564x; 1.2380x over previous
import functools

import jax
import jax.numpy as jnp
from jax.experimental import pallas as pl
from jax.experimental.pallas import tpu as pltpu


def _branch_body(adj_ref, meth_ref, w1_ref, fc1w_ref, w2_ref,
                 b1_ref, b2_ref, fc1b_ref, pool_ref,
                 h0_ref, g_ref, *, n, tm):
    x = meth_ref[0]
    h0_ref[...] = (
        jnp.dot(x, w1_ref[...], preferred_element_type=jnp.float32)
        .astype(jnp.bfloat16)
    )
    init = (
        jnp.dot(x, fc1w_ref[...], preferred_element_type=jnp.float32)
        + fc1b_ref[...]
    )
    w2 = w2_ref[...].astype(jnp.bfloat16)
    b1 = b1_ref[...]

    starts = list(range(0, n, tm))
    for t0 in starts:
        rows = min(tm, n - t0)
        a_t = adj_ref[0, t0:t0 + rows, :].astype(jnp.bfloat16)
        h1_t = jnp.maximum(
            jnp.dot(a_t, h0_ref[...], preferred_element_type=jnp.float32)
            + b1,
            0.0,
        ).astype(jnp.bfloat16)
        g_ref[t0:t0 + rows, :] = jnp.dot(
            h1_t, w2, preferred_element_type=jnp.float32
        ).astype(jnp.bfloat16)

    b2 = b2_ref[...]
    tile_maxes = []
    for t0 in starts:
        rows = min(tm, n - t0)
        a_t = adj_ref[0, t0:t0 + rows, :].astype(jnp.bfloat16)
        o_t = (
            jnp.dot(a_t, g_ref[...], preferred_element_type=jnp.float32)
            + init[t0:t0 + rows, :]
            + b2
        )
        tile_maxes.append(jnp.max(o_t, axis=0, keepdims=True))
    m = tile_maxes[0]
    for tm_ in tile_maxes[1:]:
        m = jnp.maximum(m, tm_)
    pool_ref[0] = m


def _branch(adj, meth, w1, b1, w2, b2, fc1w, fc1b):
    bsz, n, f = meth.shape
    h = w1.shape[1]
    c = w2.shape[1]
    tm = 520

    pool = pl.pallas_call(
        functools.partial(_branch_body, n=n, tm=tm),
        grid=(bsz,),
        in_specs=[
            pl.BlockSpec((1, n, n), lambda b: (b, 0, 0)),
            pl.BlockSpec((1, n, f), lambda b: (b, 0, 0)),
            pl.BlockSpec((f, h), lambda b: (0, 0)),
            pl.BlockSpec((f, c), lambda b: (0, 0)),
            pl.BlockSpec((h, c), lambda b: (0, 0)),
            pl.BlockSpec((1, h), lambda b: (0, 0)),
            pl.BlockSpec((1, c), lambda b: (0, 0)),
            pl.BlockSpec((1, c), lambda b: (0, 0)),
        ],
        out_specs=pl.BlockSpec((1, 1, c), lambda b: (b, 0, 0)),
        out_shape=jax.ShapeDtypeStruct((bsz, 1, c), jnp.float32),
        scratch_shapes=[
            pltpu.VMEM((n, h), jnp.bfloat16),
            pltpu.VMEM((n, c), jnp.bfloat16),
        ],
        compiler_params=pltpu.CompilerParams(
            dimension_semantics=("arbitrary",),
            vmem_limit_bytes=100 * 1024 * 1024,
        ),
    )(adj, meth, w1, fc1w, w2, b1, b2, fc1b)

    return pool.reshape(bsz, c)


def _mlp_body(sp_ref, vp_ref, w2a_ref, w2b_ref, b2_ref, w3_ref, b3_ref,
              w4_ref, b4_ref, w5_ref, b5_ref, out_ref):
    d = jnp.maximum(
        jnp.dot(sp_ref[...], w2a_ref[...], preferred_element_type=jnp.float32)
        + jnp.dot(vp_ref[...], w2b_ref[...], preferred_element_type=jnp.float32)
        + b2_ref[...],
        0.0,
    )
    d = jnp.maximum(
        jnp.dot(d, w3_ref[...], preferred_element_type=jnp.float32)
        + b3_ref[...],
        0.0,
    )
    d = jnp.maximum(
        jnp.dot(d, w4_ref[...], preferred_element_type=jnp.float32)
        + b4_ref[...],
        0.0,
    )
    out_ref[...] = (
        jnp.sum(d * w5_ref[...].T, axis=1, keepdims=True) + b5_ref[...]
    )


def kernel(solute_adj, solute_meth, solvent_meth, solvent_adj_meth,
           conv1_W, conv1_b, conv2_W, conv2_b,
           fc1_W, fc1_b, fc2_W, fc2_b, fc3_W, fc3_b,
           fc4_W, fc4_b, fc5_W, fc5_b):
    b1 = conv1_b.reshape(1, -1)
    b2 = conv2_b.reshape(1, -1)
    fb1 = fc1_b.reshape(1, -1)
    nclass = fc1_W.shape[1]

    su_pool = _branch(solute_adj, solute_meth, conv1_W, b1, conv2_W, b2,
                      fc1_W, fb1)
    sv_pool = _branch(solvent_adj_meth, solvent_meth, conv1_W, b1, conv2_W,
                      b2, fc1_W, fb1)

    bsz = su_pool.shape[0]
    out = pl.pallas_call(
        _mlp_body,
        out_shape=jax.ShapeDtypeStruct((bsz, 1), jnp.float32),
    )(su_pool, sv_pool,
      fc2_W[:nclass], fc2_W[nclass:], fc2_b.reshape(1, -1),
      fc3_W, fc3_b.reshape(1, -1),
      fc4_W, fc4_b.reshape(1, -1),
      fc5_W, fc5_b.reshape(1, -1))
    return out


# speedup vs baseline: 1.2953x; 1.0309x over previous
import functools

import jax
import jax.numpy as jnp
from jax.experimental import pallas as pl
from jax.experimental.pallas import tpu as pltpu


def _branch_body(adj_ref, meth_ref, w1_ref, fc1w_ref, w2_ref,
                 b1_ref, b2_ref, fc1b_ref, pool_ref,
                 abf_ref, h0_ref, g_ref, init_ref, *, n, tm, n_tiles):
    t = pl.program_id(1)

    @pl.when(t == 0)
    def _proj():
        x = meth_ref[0]
        h0_ref[...] = (
            jnp.dot(x, w1_ref[...], preferred_element_type=jnp.float32)
            .astype(jnp.bfloat16)
        )
        init_ref[0:n, :] = (
            jnp.dot(x, fc1w_ref[...], preferred_element_type=jnp.float32)
            + fc1b_ref[...]
        )

    a_t = adj_ref[0].astype(jnp.bfloat16)
    abf_ref[pl.ds(t * tm, tm), :] = a_t
    h1_t = jnp.maximum(
        jnp.dot(a_t, h0_ref[...], preferred_element_type=jnp.float32)
        + b1_ref[...],
        0.0,
    ).astype(jnp.bfloat16)
    g_ref[pl.ds(t * tm, tm), :] = jnp.dot(
        h1_t, w2_ref[...].astype(jnp.bfloat16),
        preferred_element_type=jnp.float32,
    ).astype(jnp.bfloat16)

    @pl.when(t == n_tiles - 1)
    def _pass2():
        g = g_ref[0:n, :]
        b2 = b2_ref[...]
        m = None
        for t0 in range(0, n_tiles * tm, tm):
            a2 = abf_ref[t0:t0 + tm, :]
            o = (
                jnp.dot(a2, g, preferred_element_type=jnp.float32)
                + init_ref[pl.ds(t0, tm), :]
                + b2
            )
            if t0 + tm > n:
                rows = t0 + jax.lax.broadcasted_iota(jnp.int32, o.shape, 0)
                o = jnp.where(rows < n, o, -jnp.inf)
            tmax = jnp.max(o, axis=0, keepdims=True)
            m = tmax if m is None else jnp.maximum(m, tmax)
        pool_ref[0] = m


def _branch(adj, meth, w1, b1, w2, b2, fc1w, fc1b):
    bsz, n, f = meth.shape
    h = w1.shape[1]
    c = w2.shape[1]
    tm = 520
    n_tiles = -(-n // tm)
    n_pad = n_tiles * tm

    pool = pl.pallas_call(
        functools.partial(_branch_body, n=n, tm=tm, n_tiles=n_tiles),
        grid=(bsz, n_tiles),
        in_specs=[
            pl.BlockSpec((1, tm, n), lambda b, t: (b, t, 0)),
            pl.BlockSpec((1, n, f), lambda b, t: (b, 0, 0)),
            pl.BlockSpec((f, h), lambda b, t: (0, 0)),
            pl.BlockSpec((f, c), lambda b, t: (0, 0)),
            pl.BlockSpec((h, c), lambda b, t: (0, 0)),
            pl.BlockSpec((1, h), lambda b, t: (0, 0)),
            pl.BlockSpec((1, c), lambda b, t: (0, 0)),
            pl.BlockSpec((1, c), lambda b, t: (0, 0)),
        ],
        out_specs=pl.BlockSpec((1, 1, c), lambda b, t: (b, 0, 0)),
        out_shape=jax.ShapeDtypeStruct((bsz, 1, c), jnp.float32),
        scratch_shapes=[
            pltpu.VMEM((n_pad, n), jnp.bfloat16),
            pltpu.VMEM((n, h), jnp.bfloat16),
            pltpu.VMEM((n_pad, c), jnp.bfloat16),
            pltpu.VMEM((n_pad, c), jnp.float32),
        ],
        compiler_params=pltpu.CompilerParams(
            dimension_semantics=("arbitrary", "arbitrary"),
        ),
    )(adj, meth, w1, fc1w, w2, b1, b2, fc1b)

    return pool.reshape(bsz, c)


def _mlp_body(sp_ref, vp_ref, w2a_ref, w2b_ref, b2_ref, w3_ref, b3_ref,
              w4_ref, b4_ref, w5_ref, b5_ref, out_ref):
    d = jnp.maximum(
        jnp.dot(sp_ref[...], w2a_ref[...], preferred_element_type=jnp.float32)
        + jnp.dot(vp_ref[...], w2b_ref[...], preferred_element_type=jnp.float32)
        + b2_ref[...],
        0.0,
    )
    d = jnp.maximum(
        jnp.dot(d, w3_ref[...], preferred_element_type=jnp.float32)
        + b3_ref[...],
        0.0,
    )
    d = jnp.maximum(
        jnp.dot(d, w4_ref[...], preferred_element_type=jnp.float32)
        + b4_ref[...],
        0.0,
    )
    out_ref[...] = (
        jnp.sum(d * w5_ref[...].T, axis=1, keepdims=True) + b5_ref[...]
    )


def kernel(solute_adj, solute_meth, solvent_meth, solvent_adj_meth,
           conv1_W, conv1_b, conv2_W, conv2_b,
           fc1_W, fc1_b, fc2_W, fc2_b, fc3_W, fc3_b,
           fc4_W, fc4_b, fc5_W, fc5_b):
    b1 = conv1_b.reshape(1, -1)
    b2 = conv2_b.reshape(1, -1)
    fb1 = fc1_b.reshape(1, -1)
    nclass = fc1_W.shape[1]

    su_pool = _branch(solute_adj, solute_meth, conv1_W, b1, conv2_W, b2,
                      fc1_W, fb1)
    sv_pool = _branch(solvent_adj_meth, solvent_meth, conv1_W, b1, conv2_W,
                      b2, fc1_W, fb1)

    bsz = su_pool.shape[0]
    out = pl.pallas_call(
        _mlp_body,
        out_shape=jax.ShapeDtypeStruct((bsz, 1), jnp.float32),
    )(su_pool, sv_pool,
      fc2_W[:nclass], fc2_W[nclass:], fc2_b.reshape(1, -1),
      fc3_W, fc3_b.reshape(1, -1),
      fc4_W, fc4_b.reshape(1, -1),
      fc5_W, fc5_b.reshape(1, -1))
    return out
